# baseline (speedup 1.0000x reference)
import functools

import numpy as np
import jax
import jax.numpy as jnp
from jax import lax
from jax.experimental import pallas as pl
from jax.experimental.pallas import tpu as pltpu

_STRIDE = 32.0
_AW = (116.0, 156.0, 373.0)
_AH = (90.0, 198.0, 326.0)


def _tables(f, n_ch, n_anchors):
    hw = f * f
    oc = n_anchors * n_ch
    mul = np.ones((1, oc), np.float32)
    wh = np.zeros((1, oc), np.float32)
    for a in range(n_anchors):
        mul[0, a * n_ch + 0] = _STRIDE
        mul[0, a * n_ch + 1] = _STRIDE
        mul[0, a * n_ch + 2] = _AW[a]
        mul[0, a * n_ch + 3] = _AH[a]
        wh[0, a * n_ch + 2] = 1.0
        wh[0, a * n_ch + 3] = 1.0
    add = np.zeros((hw, oc), np.float32)
    xs = np.tile(np.arange(f, dtype=np.float32), f) * _STRIDE
    ys = np.repeat(np.arange(f, dtype=np.float32), f) * _STRIDE
    for a in range(n_anchors):
        add[:, a * n_ch + 0] = xs
        add[:, a * n_ch + 1] = ys
    return mul, wh, add


def _body(x_ref, w_ref, b_ref, mul_ref, wh_ref, add_ref, o_ref, *,
          nb, hw, n_ch, n_anchors):
    w = w_ref[...]
    mul = mul_ref[...]
    wh = wh_ref[...]
    add = add_ref[...]
    for j in range(nb):
        xb = x_ref[j]
        z = lax.dot_general(xb, w, (((0,), (1,)), ((), ())),
                            preferred_element_type=jnp.float32)
        z = z + b_ref[...]
        e = jnp.exp(z)
        sig = jnp.where(z > 20.0, 1.0, e / (1.0 + e))
        base = sig + wh * (e - sig)
        out = base * mul + add
        for a in range(n_anchors):
            o_ref[j, a * hw:(a + 1) * hw, :] = out[:, a * n_ch:(a + 1) * n_ch]


def kernel(x, W, b):
    B, C, f, _ = x.shape
    n_anchors, n_ch = 3, 85
    hw = f * f
    oc = n_anchors * n_ch
    nb = 4
    b2 = b.reshape(1, oc)
    mul, wh, add = (jnp.asarray(t) for t in _tables(f, n_ch, n_anchors))

    body = functools.partial(_body, nb=nb, hw=hw, n_ch=n_ch,
                             n_anchors=n_anchors)

    def half(xh):
        bh = xh.shape[0]
        return pl.pallas_call(
            body,
            grid=(bh // nb,),
            in_specs=[
                pl.BlockSpec((nb, C, hw), lambda i: (i, 0, 0)),
                pl.BlockSpec((oc, C), lambda i: (0, 0)),
                pl.BlockSpec((1, oc), lambda i: (0, 0)),
                pl.BlockSpec((1, oc), lambda i: (0, 0)),
                pl.BlockSpec((1, oc), lambda i: (0, 0)),
                pl.BlockSpec((hw, oc), lambda i: (0, 0)),
            ],
            out_specs=pl.BlockSpec((nb, n_anchors * hw, n_ch),
                                   lambda i: (i, 0, 0)),
            out_shape=jax.ShapeDtypeStruct((bh, n_anchors * hw, n_ch),
                                           jnp.float32),
            compiler_params=pltpu.CompilerParams(
                dimension_semantics=("parallel",)),
        )(xh.reshape(bh, C, hw), W, b2, mul, wh, add)

    h = B // 2
    return jnp.concatenate([half(x[:h]), half(x[h:])], axis=0)


# speedup vs baseline: 1.6930x; 1.6930x over previous
import functools

import numpy as np
import jax
import jax.numpy as jnp
from jax import lax
from jax.experimental import pallas as pl
from jax.experimental.pallas import tpu as pltpu

_STRIDE = 32.0
_AW = (116.0, 156.0, 373.0)
_AH = (90.0, 198.0, 326.0)


def _tables(f, n_ch, n_anchors):
    hw = f * f
    oc = n_anchors * n_ch
    mul = np.ones((1, oc), np.float32)
    wh = np.zeros((1, oc), np.float32)
    for a in range(n_anchors):
        mul[0, a * n_ch + 0] = _STRIDE
        mul[0, a * n_ch + 1] = _STRIDE
        mul[0, a * n_ch + 2] = _AW[a]
        mul[0, a * n_ch + 3] = _AH[a]
        wh[0, a * n_ch + 2] = 1.0
        wh[0, a * n_ch + 3] = 1.0
    add = np.zeros((hw, oc), np.float32)
    xs = np.tile(np.arange(f, dtype=np.float32), f) * _STRIDE
    ys = np.repeat(np.arange(f, dtype=np.float32), f) * _STRIDE
    for a in range(n_anchors):
        add[:, a * n_ch + 0] = xs
        add[:, a * n_ch + 1] = ys
    return mul, wh, add


def _body(x_ref, w_ref, b_ref, mul_ref, wh_ref, add_ref, o_ref, *,
          nb, hw, n_ch, n_anchors):
    w = w_ref[...].astype(jnp.bfloat16)
    mul = mul_ref[...]
    wh = wh_ref[...]
    add = add_ref[...]
    for j in range(nb):
        xb = x_ref[j]
        z = lax.dot_general(xb, w, (((0,), (1,)), ((), ())),
                            preferred_element_type=jnp.float32)
        z = z + b_ref[...]
        e = jnp.exp(z)
        sig = jnp.where(z > 20.0, 1.0, e / (1.0 + e))
        base = sig + wh * (e - sig)
        out = base * mul + add
        for a in range(n_anchors):
            o_ref[j, a * hw:(a + 1) * hw, :] = out[:, a * n_ch:(a + 1) * n_ch]


def kernel(x, W, b):
    B, C, f, _ = x.shape
    n_anchors, n_ch = 3, 85
    hw = f * f
    oc = n_anchors * n_ch
    nb = 8
    xr = x.reshape(B, C, hw).astype(jnp.bfloat16)
    b2 = b.reshape(1, oc)
    mul, wh, add = (jnp.asarray(t) for t in _tables(f, n_ch, n_anchors))

    body = functools.partial(_body, nb=nb, hw=hw, n_ch=n_ch,
                             n_anchors=n_anchors)
    return pl.pallas_call(
        body,
        grid=(B // nb,),
        in_specs=[
            pl.BlockSpec((nb, C, hw), lambda i: (i, 0, 0)),
            pl.BlockSpec((oc, C), lambda i: (0, 0)),
            pl.BlockSpec((1, oc), lambda i: (0, 0)),
            pl.BlockSpec((1, oc), lambda i: (0, 0)),
            pl.BlockSpec((1, oc), lambda i: (0, 0)),
            pl.BlockSpec((hw, oc), lambda i: (0, 0)),
        ],
        out_specs=pl.BlockSpec((nb, n_anchors * hw, n_ch), lambda i: (i, 0, 0)),
        out_shape=jax.ShapeDtypeStruct((B, n_anchors * hw, n_ch), jnp.float32),
        compiler_params=pltpu.CompilerParams(
            dimension_semantics=("parallel",)),
    )(xr, W, b2, mul, wh, add)


# speedup vs baseline: 1.9163x; 1.1319x over previous
import functools

import numpy as np
import jax
import jax.numpy as jnp
from jax import lax
from jax.experimental import pallas as pl
from jax.experimental.pallas import tpu as pltpu

_STRIDE = 32.0
_AW = (116.0, 156.0, 373.0)
_AH = (90.0, 198.0, 326.0)


def _tables(f, n_ch, n_anchors):
    hw = f * f
    oc = n_anchors * n_ch
    mul = np.ones((1, oc), np.float32)
    wh = np.zeros((1, oc), np.float32)
    for a in range(n_anchors):
        mul[0, a * n_ch + 0] = _STRIDE
        mul[0, a * n_ch + 1] = _STRIDE
        mul[0, a * n_ch + 2] = _AW[a]
        mul[0, a * n_ch + 3] = _AH[a]
        wh[0, a * n_ch + 2] = 1.0
        wh[0, a * n_ch + 3] = 1.0
    add = np.zeros((hw, oc), np.float32)
    xs = np.tile(np.arange(f, dtype=np.float32), f) * _STRIDE
    ys = np.repeat(np.arange(f, dtype=np.float32), f) * _STRIDE
    for a in range(n_anchors):
        add[:, a * n_ch + 0] = xs
        add[:, a * n_ch + 1] = ys
    return mul, wh, add


def _body(x_ref, w_ref, b_ref, mul_ref, wh_ref, add_ref, o_ref, *,
          nb, hw, n_ch, n_anchors):
    w = w_ref[...]
    mul = mul_ref[...]
    wh = wh_ref[...]
    add = add_ref[...]
    for j in range(nb):
        xb = x_ref[j]
        z = lax.dot_general(xb, w, (((0,), (1,)), ((), ())),
                            preferred_element_type=jnp.float32)
        z = z + b_ref[...]
        e = jnp.exp(z)
        sig = jnp.where(z > 20.0, 1.0, e / (1.0 + e))
        base = jnp.where(wh > 0.0, e, sig)
        out = base * mul + add
        for a in range(n_anchors):
            o_ref[j, a * hw:(a + 1) * hw, :] = out[:, a * n_ch:(a + 1) * n_ch]


def kernel(x, W, b):
    B, C, f, _ = x.shape
    n_anchors, n_ch = 3, 85
    hw = f * f
    oc = n_anchors * n_ch
    nb = 8
    xr = x.reshape(B, C, hw)
    b2 = b.reshape(1, oc)
    mul, wh, add = (jnp.asarray(t) for t in _tables(f, n_ch, n_anchors))

    body = functools.partial(_body, nb=nb, hw=hw, n_ch=n_ch,
                             n_anchors=n_anchors)
    return pl.pallas_call(
        body,
        grid=(B // nb,),
        in_specs=[
            pl.BlockSpec((nb, C, hw), lambda i: (i, 0, 0)),
            pl.BlockSpec((oc, C), lambda i: (0, 0)),
            pl.BlockSpec((1, oc), lambda i: (0, 0)),
            pl.BlockSpec((1, oc), lambda i: (0, 0)),
            pl.BlockSpec((1, oc), lambda i: (0, 0)),
            pl.BlockSpec((hw, oc), lambda i: (0, 0)),
        ],
        out_specs=pl.BlockSpec((nb, n_anchors * hw, n_ch), lambda i: (i, 0, 0)),
        out_shape=jax.ShapeDtypeStruct((B, n_anchors * hw, n_ch), jnp.float32),
        compiler_params=pltpu.CompilerParams(
            dimension_semantics=("parallel",)),
    )(xr, W, b2, mul, wh, add)


# speedup vs baseline: 1.9179x; 1.0009x over previous
import functools
import math

import numpy as np
import jax
import jax.numpy as jnp
from jax import lax
from jax.experimental import pallas as pl
from jax.experimental.pallas import tpu as pltpu

_STRIDE = 32.0
_AW = (116.0, 156.0, 373.0)
_AH = (90.0, 198.0, 326.0)


def _tables(f, n_ch, n_anchors):
    hw = f * f
    oc = n_anchors * n_ch
    mul = np.ones((1, oc), np.float32)
    wh = np.zeros((1, oc), np.float32)
    for a in range(n_anchors):
        mul[0, a * n_ch + 0] = _STRIDE
        mul[0, a * n_ch + 1] = _STRIDE
        mul[0, a * n_ch + 2] = _AW[a]
        mul[0, a * n_ch + 3] = _AH[a]
        wh[0, a * n_ch + 2] = 1.0
        wh[0, a * n_ch + 3] = 1.0
    add = np.zeros((hw, oc), np.float32)
    xs = np.tile(np.arange(f, dtype=np.float32), f) * _STRIDE
    ys = np.repeat(np.arange(f, dtype=np.float32), f) * _STRIDE
    for a in range(n_anchors):
        add[:, a * n_ch + 0] = xs
        add[:, a * n_ch + 1] = ys
    return mul, wh, add


def _body(x_ref, w_ref, b_ref, mul_ref, wh_ref, add_ref, o_ref, *,
          nb, hw, n_ch, n_anchors):
    w = w_ref[...]
    mul = mul_ref[...]
    wh = wh_ref[...]
    add = add_ref[...]
    for j in range(nb):
        xb = x_ref[j]
        z = lax.dot_general(xb, w, (((0,), (1,)), ((), ())),
                            preferred_element_type=jnp.float32)
        z = z + b_ref[...]
        e = jnp.exp(z)
        sig = jnp.where(z > 20.0, 1.0, e / (1.0 + e))
        base = jnp.where(wh > 0.0, e, sig)
        out = base * mul + add
        for a in range(n_anchors):
            o_ref[j, a * hw:(a + 1) * hw, :] = out[:, a * n_ch:(a + 1) * n_ch]


def kernel(x, W, b):
    B, C, f, _ = x.shape
    n_anchors, n_ch = 3, 85
    hw = f * f
    oc = n_anchors * n_ch
    nb = math.gcd(B, 8)
    xr = x.reshape(B, C, hw)
    b2 = b.reshape(1, oc)
    mul, wh, add = (jnp.asarray(t) for t in _tables(f, n_ch, n_anchors))

    body = functools.partial(_body, nb=nb, hw=hw, n_ch=n_ch,
                             n_anchors=n_anchors)
    return pl.pallas_call(
        body,
        grid=(B // nb,),
        in_specs=[
            pl.BlockSpec((nb, C, hw), lambda i: (i, 0, 0)),
            pl.BlockSpec((oc, C), lambda i: (0, 0)),
            pl.BlockSpec((1, oc), lambda i: (0, 0)),
            pl.BlockSpec((1, oc), lambda i: (0, 0)),
            pl.BlockSpec((1, oc), lambda i: (0, 0)),
            pl.BlockSpec((hw, oc), lambda i: (0, 0)),
        ],
        out_specs=pl.BlockSpec((nb, n_anchors * hw, n_ch), lambda i: (i, 0, 0)),
        out_shape=jax.ShapeDtypeStruct((B, n_anchors * hw, n_ch), jnp.float32),
        compiler_params=pltpu.CompilerParams(
            dimension_semantics=("parallel",)),
    )(xr, W, b2, mul, wh, add)
